# baseline (speedup 1.0000x reference)
import functools

import jax
import jax.numpy as jnp
from jax import lax
from jax.experimental import pallas as pl
from jax.experimental.pallas import tpu as pltpu
from jax.experimental.pallas import tpu_sc as plsc

E = 160000
D = 256
NLANE = 16
NW = 32
CHUNK = 64
ROWS = 2 * CHUNK
EPW = 5120
E_PAD = NW * EPW
NCHUNK_FULL = EPW // CHUNK
LAST_COUNT = E - 31 * EPW
NCHUNK_LAST = LAST_COUNT // CHUNK
NGROUP = CHUNK // NLANE

_mesh = plsc.VectorSubcoreMesh(core_axis_name="c", subcore_axis_name="s")


@functools.partial(
    pl.kernel,
    out_type=jax.ShapeDtypeStruct((E,), jnp.float32),
    mesh=_mesh,
    compiler_params=pltpu.CompilerParams(needs_layout_passes=False),
    scratch_types=[
        pltpu.VMEM((2 * EPW,), jnp.int32),
        pltpu.VMEM((ROWS, D), jnp.float32),
        pltpu.VMEM((ROWS, D), jnp.float32),
        pltpu.VMEM((EPW,), jnp.float32),
        pltpu.VMEM((NLANE * NLANE,), jnp.float32),
        pltpu.SemaphoreType.DMA,
        pltpu.SemaphoreType.DMA,
    ],
)
def _decode(z_hbm, cidx_hbm, out_hbm,
            cidx, rows0, rows1, obuf, ttile, sem0, sem1):
    wid = lax.axis_index("s") * 2 + lax.axis_index("c")
    base = wid * EPW
    nchunk = jnp.where(wid == NW - 1, NCHUNK_LAST, NCHUNK_FULL)

    pltpu.sync_copy(cidx_hbm.at[pl.ds(2 * base, 2 * EPW)], cidx)

    lane = lax.iota(jnp.int32, NLANE)
    bufs = ((rows0, sem0), (rows1, sem1))

    def gather_copy(c, b):
        buf, sem = bufs[b]
        return pltpu.make_async_copy(
            z_hbm.at[cidx.at[pl.ds(c * ROWS, ROWS)]], buf, sem)

    def compute(c, b):
        buf, _ = bufs[b]
        off = c * CHUNK
        for g in range(NGROUP):
            gbase = g * NLANE
            for e in range(NLANE):
                row = gbase + e
                acc = (buf[row, pl.ds(0, NLANE)]
                       * buf[CHUNK + row, pl.ds(0, NLANE)])
                for k in range(1, D // NLANE):
                    acc = acc + (buf[row, pl.ds(k * NLANE, NLANE)]
                                 * buf[CHUNK + row, pl.ds(k * NLANE, NLANE)])
                plsc.store_scatter(ttile, [e * NLANE + ((lane + e) & 15)], acc)
            res = plsc.load_gather(ttile, [lane * NLANE + (lane & 15)])
            for col in range(1, NLANE):
                res = res + plsc.load_gather(
                    ttile, [lane * NLANE + ((lane + col) & 15)])
            obuf[pl.ds(off + gbase, NLANE)] = 1.0 / (1.0 + jnp.exp(-res))

    gather_copy(0, 0).start()

    def pair_body(cc, carry):
        c0 = 2 * cc
        c1 = c0 + 1
        gather_copy(c1, 1).start()
        gather_copy(c0, 0).wait()
        compute(c0, 0)

        @pl.when(c1 + 1 < nchunk)
        def _():
            gather_copy(c1 + 1, 0).start()

        gather_copy(c1, 1).wait()
        compute(c1, 1)
        return carry

    lax.fori_loop(0, nchunk // 2, pair_body, 0)

    @pl.when(wid < NW - 1)
    def _():
        pltpu.sync_copy(obuf.at[pl.ds(0, EPW)], out_hbm.at[pl.ds(base, EPW)])

    @pl.when(wid == NW - 1)
    def _():
        pltpu.sync_copy(obuf.at[pl.ds(0, LAST_COUNT)],
                        out_hbm.at[pl.ds((NW - 1) * EPW, LAST_COUNT)])


def kernel(z, edge_index):
    src = jnp.pad(edge_index[0].astype(jnp.int32), (0, E_PAD - E))
    dst = jnp.pad(edge_index[1].astype(jnp.int32), (0, E_PAD - E))
    nchunks = E_PAD // CHUNK
    comb = jnp.stack(
        [src.reshape(nchunks, CHUNK), dst.reshape(nchunks, CHUNK)], axis=1
    ).reshape(-1)
    return _decode(z, comb)


# speedup vs baseline: 2.2305x; 2.2305x over previous
import functools

import jax
import jax.numpy as jnp
from jax import lax
from jax.experimental import pallas as pl
from jax.experimental.pallas import tpu as pltpu
from jax.experimental.pallas import tpu_sc as plsc

E = 160000
D = 256
NLANE = 16
NW = 32
CHUNK = 64
ROWS = 2 * CHUNK
EPW = 5120
E_PAD = NW * EPW
NCHUNK_FULL = EPW // CHUNK
LAST_COUNT = E - 31 * EPW
NCHUNK_LAST = LAST_COUNT // CHUNK
NGROUP = CHUNK // NLANE

_mesh = plsc.VectorSubcoreMesh(core_axis_name="c", subcore_axis_name="s")


@functools.partial(
    pl.kernel,
    out_type=jax.ShapeDtypeStruct((E,), jnp.float32),
    mesh=_mesh,
    compiler_params=pltpu.CompilerParams(needs_layout_passes=False),
    scratch_types=[
        pltpu.VMEM((2 * EPW,), jnp.int32),
        pltpu.VMEM((ROWS, D), jnp.float32),
        pltpu.VMEM((ROWS, D), jnp.float32),
        pltpu.VMEM((EPW,), jnp.float32),
        pltpu.VMEM((NLANE * NLANE,), jnp.float32),
        pltpu.SemaphoreType.DMA,
        pltpu.SemaphoreType.DMA,
    ],
)
def _decode(z_hbm, cidx_hbm, out_hbm,
            cidx, rows0, rows1, obuf, ttile, sem0, sem1):
    wid = lax.axis_index("s") * 2 + lax.axis_index("c")
    base = wid * EPW
    nchunk = jnp.where(wid == NW - 1, NCHUNK_LAST, NCHUNK_FULL)

    pltpu.sync_copy(cidx_hbm.at[pl.ds(2 * base, 2 * EPW)], cidx)

    lane = lax.iota(jnp.int32, NLANE)
    bufs = ((rows0, sem0), (rows1, sem1))

    def gather_copy(c, b):
        buf, sem = bufs[b]
        return pltpu.make_async_copy(
            z_hbm.at[cidx.at[pl.ds(c * ROWS, ROWS)]], buf, sem)

    def compute(c, b):
        buf, _ = bufs[b]
        off = c * CHUNK

        def group_body(g, carry):
            gbase = g * NLANE
            for e in range(NLANE):
                row = gbase + e
                acc = (buf[row, pl.ds(0, NLANE)]
                       * buf[CHUNK + row, pl.ds(0, NLANE)])
                for k in range(1, D // NLANE):
                    acc = acc + (buf[row, pl.ds(k * NLANE, NLANE)]
                                 * buf[CHUNK + row, pl.ds(k * NLANE, NLANE)])
                plsc.store_scatter(ttile, [e * NLANE + ((lane + e) & 15)], acc)
            res = plsc.load_gather(ttile, [lane * NLANE + (lane & 15)])
            for col in range(1, NLANE):
                res = res + plsc.load_gather(
                    ttile, [lane * NLANE + ((lane + col) & 15)])
            obuf[pl.ds(off + gbase, NLANE)] = 1.0 / (1.0 + jnp.exp(-res))
            return carry

        lax.fori_loop(0, NGROUP, group_body, 0)

    gather_copy(0, 0).start()

    def pair_body(cc, carry):
        c0 = 2 * cc
        c1 = c0 + 1
        gather_copy(c1, 1).start()
        gather_copy(c0, 0).wait()
        compute(c0, 0)

        @pl.when(c1 + 1 < nchunk)
        def _():
            gather_copy(c1 + 1, 0).start()

        gather_copy(c1, 1).wait()
        compute(c1, 1)
        return carry

    lax.fori_loop(0, nchunk // 2, pair_body, 0)

    @pl.when(wid < NW - 1)
    def _():
        pltpu.sync_copy(obuf.at[pl.ds(0, EPW)], out_hbm.at[pl.ds(base, EPW)])

    @pl.when(wid == NW - 1)
    def _():
        pltpu.sync_copy(obuf.at[pl.ds(0, LAST_COUNT)],
                        out_hbm.at[pl.ds((NW - 1) * EPW, LAST_COUNT)])


def kernel(z, edge_index):
    src = jnp.pad(edge_index[0].astype(jnp.int32), (0, E_PAD - E))
    dst = jnp.pad(edge_index[1].astype(jnp.int32), (0, E_PAD - E))
    nchunks = E_PAD // CHUNK
    comb = jnp.stack(
        [src.reshape(nchunks, CHUNK), dst.reshape(nchunks, CHUNK)], axis=1
    ).reshape(-1)
    return _decode(z, comb)
